# baseline (speedup 1.0000x reference)
import functools

import jax
import jax.numpy as jnp
from jax import lax
from jax.experimental import pallas as pl
from jax.experimental.pallas import tpu as pltpu
from jax.experimental.pallas import tpu_sc as plsc

_VOCAB = 100000
_ROWS = 128
_STRIDE = 1000
_NUM_ALLOWED = 64
_CHUNK = 10000
_NCHUNK = _VOCAB // _CHUNK
_NEG_INF = float("-inf")


def _sc_gather(scores_t):
    mesh = plsc.VectorSubcoreMesh(core_axis_name="c", subcore_axis_name="s")

    @functools.partial(
        pl.kernel,
        out_type=jax.ShapeDtypeStruct((_NUM_ALLOWED * _ROWS,), jnp.float32),
        mesh=mesh,
        scratch_types=[pltpu.VMEM((_ROWS,), jnp.float32)],
        compiler_params=pltpu.CompilerParams(needs_layout_passes=False),
    )
    def k(s_hbm, o_hbm, row_v):
        wid = lax.axis_index("c") * 16 + lax.axis_index("s")
        for t in range(2):
            c = wid * 2 + t
            src_row = pl.multiple_of(c * _STRIDE, 8)
            dst_off = pl.multiple_of(c * _ROWS, 8)
            pltpu.sync_copy(s_hbm.at[src_row], row_v)
            pltpu.sync_copy(row_v, o_hbm.at[pl.ds(dst_off, _ROWS)])

    return k(scores_t)


def _tc_body(g_ref, o_ref, const_ref, fill_sems, ins_sem):
    const_ref[...] = jnp.full(const_ref.shape, _NEG_INF, jnp.float32)

    fills = []
    for b in range(_NCHUNK):
        cp = pltpu.make_async_copy(
            const_ref, o_ref.at[pl.ds(b * _CHUNK, _CHUNK), :], fill_sems.at[b]
        )
        cp.start()
        fills.append(cp)

    inserts = []
    for b in range(_NCHUNK):
        fills[b].wait()
        lo = -(-b * _CHUNK // _STRIDE)
        hi = min(_NUM_ALLOWED, ((b + 1) * _CHUNK - 1) // _STRIDE + 1)
        for c in range(lo, hi):
            cp = pltpu.make_async_copy(
                g_ref.at[pl.ds(c * _ROWS, _ROWS)],
                o_ref.at[c * _STRIDE],
                ins_sem,
            )
            cp.start()
            inserts.append(cp)

    for cp in inserts:
        cp.wait()


def kernel(input_ids, scores):
    del input_ids
    scores_t = scores.T
    gathered = _sc_gather(scores_t)
    out_t = pl.pallas_call(
        _tc_body,
        grid=(1,),
        in_specs=[pl.BlockSpec(memory_space=pl.ANY)],
        out_specs=pl.BlockSpec(memory_space=pl.ANY),
        out_shape=jax.ShapeDtypeStruct((_VOCAB, _ROWS), jnp.float32),
        scratch_shapes=[
            pltpu.VMEM((_CHUNK, _ROWS), jnp.float32),
            pltpu.SemaphoreType.DMA((_NCHUNK,)),
            pltpu.SemaphoreType.DMA,
        ],
    )(gathered)
    return out_t.T
